# baseline (speedup 1.0000x reference)
import jax
import jax.numpy as jnp
from jax.experimental import pallas as pl
from jax.experimental.pallas import tpu as pltpu

_B = 4096
_D = 128
_N = 2 * _B
_TEMP = 0.1
_BR = 2048
_NBLK = _N // _BR
_NPAIRS = _NBLK * (_NBLK + 1) // 2

_LOG2E = 1.4426950408889634
_C = (_LOG2E / _TEMP) ** 0.5
_LN2 = 0.6931471805599453


def _ntxent_kernel(zi_ref, zj_ref, out_ref, rn_ref, acc_ref):
    k = pl.program_id(0)

    @pl.when(k == 0)
    def _init():
        r = jnp.concatenate([zi_ref[...], zj_ref[...]], axis=0)
        nrm = jnp.maximum(jnp.sqrt(jnp.sum(r * r, axis=1, keepdims=True)), 1e-12)
        rn_ref[...] = r * (_C / nrm)
        acc_ref[...] = jnp.zeros_like(acc_ref)

    i_blk = jnp.where(k < 4, 0, jnp.where(k < 7, 1, jnp.where(k < 9, 2, 3)))
    base = jnp.where(k < 4, 0, jnp.where(k < 7, 4, jnp.where(k < 9, 7, 9)))
    j_blk = k - base + i_blk

    ri = rn_ref[pl.ds(i_blk * _BR, _BR), :]
    rj = rn_ref[pl.ds(j_blk * _BR, _BR), :]
    s2 = jax.lax.dot_general(
        ri, rj, (((1,), (1,)), ((), ())),
        preferred_element_type=jnp.float32,
    )
    e = jnp.exp2(s2)
    ones = jnp.ones((_BR, 1), dtype=jnp.float32)
    rowsum = jax.lax.dot_general(
        e, ones, (((1,), (0,)), ((), ())), preferred_element_type=jnp.float32)
    acc_ref[pl.ds(i_blk * _BR, _BR), :] += rowsum

    @pl.when(i_blk != j_blk)
    def _colsum():
        colsum = jax.lax.dot_general(
            e, ones, (((0,), (0,)), ((), ())), preferred_element_type=jnp.float32)
        acc_ref[pl.ds(j_blk * _BR, _BR), :] += colsum

    @pl.when(k == _NPAIRS - 1)
    def _epilogue():
        rn = rn_ref[...]
        selfdot = jnp.sum(rn * rn, axis=1, keepdims=True)
        tot = acc_ref[...] - jnp.exp2(selfdot)
        lse_sum = jnp.sum(jnp.log(tot))
        pos_sum = jnp.sum(rn_ref[0:_B, :] * rn_ref[_B:_N, :])
        out_ref[0, 0] = (lse_sum - 2.0 * _LN2 * pos_sum) * (1.0 / _N)


def kernel(z_i, z_j):
    out = pl.pallas_call(
        _ntxent_kernel,
        grid=(_NPAIRS,),
        in_specs=[
            pl.BlockSpec((_B, _D), lambda k: (0, 0)),
            pl.BlockSpec((_B, _D), lambda k: (0, 0)),
        ],
        out_specs=pl.BlockSpec(memory_space=pltpu.SMEM),
        out_shape=jax.ShapeDtypeStruct((1, 1), jnp.float32),
        scratch_shapes=[
            pltpu.VMEM((_N, _D), jnp.float32),
            pltpu.VMEM((_N, 1), jnp.float32),
        ],
    )(z_i, z_j)
    return out[0, 0]


# speedup vs baseline: 1.8226x; 1.8226x over previous
import jax
import jax.numpy as jnp
from jax.experimental import pallas as pl
from jax.experimental.pallas import tpu as pltpu

_B = 4096
_D = 128
_N = 2 * _B
_TEMP = 0.1
_BR = 2048
_NBLK = _N // _BR
_NPAIRS = _NBLK * (_NBLK + 1) // 2

_LOG2E = 1.4426950408889634
_C = (_LOG2E / _TEMP) ** 0.5
_LN2 = 0.6931471805599453


def _ntxent_kernel(zi_ref, zj_ref, out_ref, rn_ref, acc_ref):
    k = pl.program_id(0)

    @pl.when(k == 0)
    def _init():
        r = jnp.concatenate([zi_ref[...], zj_ref[...]], axis=0)
        nrm = jnp.maximum(jnp.sqrt(jnp.sum(r * r, axis=1, keepdims=True)), 1e-12)
        rn_ref[...] = r * (_C / nrm)
        acc_ref[...] = jnp.zeros_like(acc_ref)

    i_blk = jnp.where(k < 4, 0, jnp.where(k < 7, 1, jnp.where(k < 9, 2, 3)))
    base = jnp.where(k < 4, 0, jnp.where(k < 7, 4, jnp.where(k < 9, 7, 9)))
    j_blk = k - base + i_blk

    ri = rn_ref[pl.ds(i_blk * _BR, _BR), :]
    rj = rn_ref[pl.ds(j_blk * _BR, _BR), :]
    s2 = jax.lax.dot_general(
        ri, rj, (((1,), (1,)), ((), ())),
        preferred_element_type=jnp.float32,
    )
    e = jnp.exp2(s2)
    rowsum = jnp.sum(e, axis=1, keepdims=True)
    acc_ref[pl.ds(i_blk * _BR, _BR), :] += rowsum

    @pl.when(i_blk != j_blk)
    def _colsum():
        colsum = jnp.sum(e, axis=0, keepdims=True)
        acc_ref[pl.ds(j_blk * _BR, _BR), :] += jnp.transpose(colsum, (1, 0))

    @pl.when(k == _NPAIRS - 1)
    def _epilogue():
        rn = rn_ref[...]
        selfdot = jnp.sum(rn * rn, axis=1, keepdims=True)
        tot = acc_ref[...] - jnp.exp2(selfdot)
        lse_sum = jnp.sum(jnp.log(tot))
        pos_sum = jnp.sum(rn_ref[0:_B, :] * rn_ref[_B:_N, :])
        out_ref[0, 0] = (lse_sum - 2.0 * _LN2 * pos_sum) * (1.0 / _N)


def kernel(z_i, z_j):
    out = pl.pallas_call(
        _ntxent_kernel,
        grid=(_NPAIRS,),
        in_specs=[
            pl.BlockSpec((_B, _D), lambda k: (0, 0)),
            pl.BlockSpec((_B, _D), lambda k: (0, 0)),
        ],
        out_specs=pl.BlockSpec(memory_space=pltpu.SMEM),
        out_shape=jax.ShapeDtypeStruct((1, 1), jnp.float32),
        scratch_shapes=[
            pltpu.VMEM((_N, _D), jnp.float32),
            pltpu.VMEM((_N, 1), jnp.float32),
        ],
    )(z_i, z_j)
    return out[0, 0]
